# baseline (speedup 1.0000x reference)
import functools

import jax
import jax.numpy as jnp
from jax.experimental import pallas as pl

_TEMP = 20.0
_EPS = 1e-12
_BN = 36864


def _tile_kernel(feat_ref, mem_ref, out_ref):
    feat = feat_ref[...]
    norm = jnp.sqrt(jnp.sum(feat * feat, axis=-1, keepdims=True))
    feat_n = feat / jnp.maximum(norm, _EPS)
    out_ref[...] = jax.lax.dot_general(
        feat_n.astype(jnp.bfloat16),
        mem_ref[...].astype(jnp.bfloat16),
        dimension_numbers=(((1,), (1,)), ((), ())),
        preferred_element_type=jnp.float32,
    ) * _TEMP


@functools.partial(jax.jit, static_argnames=())
def kernel(feat, index, memory):
    del index
    batch, feat_dim = feat.shape
    n_data = memory.shape[0]
    grid = (pl.cdiv(n_data, _BN),)
    return pl.pallas_call(
        _tile_kernel,
        grid=grid,
        in_specs=[
            pl.BlockSpec((batch, feat_dim), lambda i: (0, 0)),
            pl.BlockSpec((_BN, feat_dim), lambda i: (i, 0)),
        ],
        out_specs=pl.BlockSpec((batch, _BN), lambda i: (0, i)),
        out_shape=jax.ShapeDtypeStruct((batch, n_data), jnp.float32),
    )(feat, memory)


# speedup vs baseline: 1.0094x; 1.0094x over previous
import functools

import jax
import jax.numpy as jnp
from jax.experimental import pallas as pl
from jax.experimental.pallas import tpu as pltpu

_TEMP = 20.0
_EPS = 1e-12
_BN = 32768


def _tile_kernel(feat_ref, mem_ref, out_ref):
    feat = feat_ref[...]
    norm = jnp.sqrt(jnp.sum(feat * feat, axis=-1, keepdims=True))
    feat_n = feat / jnp.maximum(norm, _EPS)
    out_ref[...] = jax.lax.dot_general(
        feat_n.astype(jnp.bfloat16),
        mem_ref[...].astype(jnp.bfloat16),
        dimension_numbers=(((1,), (1,)), ((), ())),
        preferred_element_type=jnp.float32,
    ) * _TEMP


@functools.partial(jax.jit, static_argnames=())
def kernel(feat, index, memory):
    del index
    batch, feat_dim = feat.shape
    n_data = memory.shape[0]
    grid = (pl.cdiv(n_data, _BN),)
    return pl.pallas_call(
        _tile_kernel,
        grid=grid,
        in_specs=[
            pl.BlockSpec((batch, feat_dim), lambda i: (0, 0)),
            pl.BlockSpec((_BN, feat_dim), lambda i: (i, 0)),
        ],
        out_specs=pl.BlockSpec((batch, _BN), lambda i: (0, i)),
        out_shape=jax.ShapeDtypeStruct((batch, n_data), jnp.float32),
        compiler_params=pltpu.CompilerParams(
            dimension_semantics=("parallel",),
        ),
    )(feat, memory)
